# baseline (speedup 1.0000x reference)
import jax
import jax.numpy as jnp
from jax import lax
from jax.experimental import pallas as pl
from jax.experimental.pallas import tpu as pltpu
from jax.experimental.pallas import tpu_sc as plsc

N = 4194304
C = 10
L = 16
NC = 2
NS = 16
NW = NC * NS
NBLK = N // 128
STATS = 4
NBANK = 4

B_TC = 3072
B_SC = NBLK - B_TC
PBLK = B_SC // NW
TB = 32
NCHUNK = PBLK // TB
T = TB * 128
TBC = 32
NSTEP = B_TC // TBC


def _sc_stats_body(feat_hbm, ys_hbm, out_hbm, xy0, xy1, ys0, ys1,
                   acc_c, acc_x, acc_y, acc_n, pub, sem0, sem1):
    wid = lax.axis_index("s") * NC + lax.axis_index("c")
    base_blk = B_TC + wid * PBLK
    base = base_blk * 128
    z = jnp.zeros((L,), jnp.float32)
    for r in range(C * NBANK):
        acc_c[pl.ds(r * L, L)] = z
        acc_x[pl.ds(r * L, L)] = z
        acc_y[pl.ds(r * L, L)] = z
        acc_n[pl.ds(r * L, L)] = z
    lane = lax.iota(jnp.int32, L)
    lane_b = [lane + k * (C * L) for k in range(NBANK)]
    ones = jnp.ones((L,), jnp.float32)

    def issue(xy_v, ys_v, sem, g):
        pltpu.async_copy(feat_hbm.at[pl.ds(base_blk + g * TB, TB)], xy_v, sem)
        pltpu.async_copy(ys_hbm.at[pl.ds(base + g * T, T)], ys_v, sem)

    def wait(xy_v, ys_v, sem):
        pltpu.make_async_copy(feat_hbm.at[pl.ds(0, TB)], xy_v, sem).wait()
        pltpu.make_async_copy(ys_hbm.at[pl.ds(0, T)], ys_v, sem).wait()

    def compute(xy_v, ys_v):
        @plsc.parallel_loop(0, TB, unroll=2)
        def _blk(b):
            for grp in range(8):
                ys_f = ys_v[pl.ds(b * 128 + grp * L, L)]
                cls = ys_f.astype(jnp.int32)
                xv = xy_v[b, 0, pl.ds(grp * L, L)]
                yv = xy_v[b, 1, pl.ds(grp * L, L)]
                nv = xv * xv + yv * yv
                sidx = cls * L + lane_b[grp % NBANK]
                plsc.addupdate_scatter(acc_c, [sidx], ones)
                plsc.addupdate_scatter(acc_x, [sidx], xv)
                plsc.addupdate_scatter(acc_y, [sidx], yv)
                plsc.addupdate_scatter(acc_n, [sidx], nv)

    issue(xy0, ys0, sem0, 0)
    NPAIR = NCHUNK - (NCHUNK % 2)

    @pl.loop(0, NPAIR, step=2)
    def _chunk(g):
        @pl.when(g + 1 < NCHUNK)
        def _():
            issue(xy1, ys1, sem1, g + 1)
        wait(xy0, ys0, sem0)
        compute(xy0, ys0)

        @pl.when(g + 2 < NCHUNK)
        def _():
            issue(xy0, ys0, sem0, g + 2)
        wait(xy1, ys1, sem1)
        compute(xy1, ys1)

    if NCHUNK % 2:
        wait(xy0, ys0, sem0)
        compute(xy0, ys0)

    for k, acc in enumerate((acc_c, acc_x, acc_y, acc_n)):
        for r in range(C):
            s = pl.ds(r * L, L)
            v = acc[s]
            for q in range(1, NBANK):
                v = v + acc[pl.ds(q * C * L + r * L, L)]
            pub[s] = v
        pltpu.sync_copy(pub, out_hbm.at[wid, k])


def _run_sc(feat_blk, ys):
    mesh = plsc.VectorSubcoreMesh(core_axis_name="c", subcore_axis_name="s")
    kfn = pl.kernel(
        _sc_stats_body,
        out_type=jax.ShapeDtypeStruct((NW, STATS, C * L), jnp.float32),
        mesh=mesh,
        compiler_params=pltpu.CompilerParams(needs_layout_passes=False),
        scratch_types=[
            pltpu.VMEM((TB, 2, 128), jnp.float32),
            pltpu.VMEM((TB, 2, 128), jnp.float32),
            pltpu.VMEM((T,), jnp.float32),
            pltpu.VMEM((T,), jnp.float32),
            pltpu.VMEM((NBANK * C * L,), jnp.float32),
            pltpu.VMEM((NBANK * C * L,), jnp.float32),
            pltpu.VMEM((NBANK * C * L,), jnp.float32),
            pltpu.VMEM((NBANK * C * L,), jnp.float32),
            pltpu.VMEM((C * L,), jnp.float32),
            pltpu.SemaphoreType.DMA,
            pltpu.SemaphoreType.DMA,
        ],
    )
    return kfn(feat_blk, ys)


def _tc_stats_body(feat_ref, ys_ref, out_ref):
    i = pl.program_id(0)

    @pl.when(i == 0)
    def _():
        out_ref[...] = jnp.zeros((STATS, C * 128), jnp.float32)

    xb = feat_ref[:, 0, :]
    yb = feat_ref[:, 1, :]
    ysb = ys_ref[...]
    nb = xb * xb + yb * yb
    onesrow = jnp.ones((1, TBC), jnp.float32)
    for c in range(C):
        m = (ysb == float(c)).astype(jnp.float32)
        sl = pl.ds(c * 128, 128)
        out_ref[0:1, sl] += jnp.dot(onesrow, m,
                                    preferred_element_type=jnp.float32)
        out_ref[1:2, sl] += jnp.dot(onesrow, m * xb,
                                    preferred_element_type=jnp.float32)
        out_ref[2:3, sl] += jnp.dot(onesrow, m * yb,
                                    preferred_element_type=jnp.float32)
        out_ref[3:4, sl] += jnp.dot(onesrow, m * nb,
                                    preferred_element_type=jnp.float32)


def _run_tc(feat_blk, ys2):
    return pl.pallas_call(
        _tc_stats_body,
        grid=(NSTEP,),
        in_specs=[
            pl.BlockSpec((TBC, 2, 128), lambda i: (i, 0, 0)),
            pl.BlockSpec((TBC, 128), lambda i: (i, 0)),
        ],
        out_specs=pl.BlockSpec((STATS, C * 128), lambda i: (0, 0)),
        out_shape=jax.ShapeDtypeStruct((STATS, C * 128), jnp.float32),
    )(feat_blk, ys2)


def _fin_body(sc_ref, tc_ref, m_ref, m3_ref, ct_ref, scale_ref, out_ref):
    red = jnp.dot(jnp.sum(sc_ref[...], axis=0), m_ref[...],
                  preferred_element_type=jnp.float32)
    red = red + jnp.dot(tc_ref[...], m3_ref[...],
                        preferred_element_type=jnp.float32)
    cnt, sx, sy, sn = red[0:1], red[1:2], red[2:3], red[3:4]
    cx, cy = ct_ref[0:1], ct_ref[1:2]
    term = sn - 2.0 * (cx * sx + cy * sy) + cnt * (cx * cx + cy * cy)
    safe = jnp.where(cnt > 0, cnt, 1.0)
    per = jnp.where(cnt > 0, term / safe, 0.0)
    out_ref[...] = jnp.sum(per, axis=1, keepdims=True) * scale_ref[...]


def kernel(features, ys, center, lambdas):
    feat_blk = features.reshape(NBLK, 128, 2).transpose(0, 2, 1)
    ys2 = ys.reshape(NBLK, 128)
    sc_stats = _run_sc(feat_blk, ys)
    if B_TC:
        tc_stats = _run_tc(feat_blk, ys2)
    else:
        tc_stats = jnp.zeros((STATS, C * 128), jnp.float32)
    m = jnp.kron(jnp.eye(C, dtype=jnp.float32),
                 jnp.ones((L, 1), jnp.float32))
    m3 = jnp.kron(jnp.eye(C, dtype=jnp.float32),
                  jnp.ones((128, 1), jnp.float32))
    ct = center.T
    scale = (jnp.asarray(lambdas, jnp.float32) / 2.0 / N).reshape(1, 1)
    out = pl.pallas_call(
        _fin_body,
        out_shape=jax.ShapeDtypeStruct((1, 1), jnp.float32),
    )(sc_stats, tc_stats, m, m3, ct, scale)
    return out[0, 0]


# speedup vs baseline: 1.1630x; 1.1630x over previous
import jax
import jax.numpy as jnp
from jax import lax
from jax.experimental import pallas as pl
from jax.experimental.pallas import tpu as pltpu
from jax.experimental.pallas import tpu_sc as plsc

N = 4194304
C = 10
L = 16
NC = 2
NS = 16
NW = NC * NS
NBLK = N // 128
STATS = 4
NBANK = 4

B_TC = 2048
B_SC = NBLK - B_TC
PBLK = B_SC // NW
TB = 64
NCHUNK = PBLK // TB
T = TB * 128
TBC = 32
NSTEP = B_TC // TBC


def _sc_stats_body(feat_hbm, ys_hbm, out_hbm, xy0, xy1, ys0, ys1,
                   acc_c, acc_x, acc_y, acc_n, pub, sem0, sem1):
    wid = lax.axis_index("s") * NC + lax.axis_index("c")
    base_blk = B_TC + wid * PBLK
    base = base_blk * 128
    z = jnp.zeros((L,), jnp.float32)
    for r in range(C * NBANK):
        acc_c[pl.ds(r * L, L)] = z
        acc_x[pl.ds(r * L, L)] = z
        acc_y[pl.ds(r * L, L)] = z
        acc_n[pl.ds(r * L, L)] = z
    lane = lax.iota(jnp.int32, L)
    lane_b = [lane + k * (C * L) for k in range(NBANK)]
    ones = jnp.ones((L,), jnp.float32)

    def issue(xy_v, ys_v, sem, g):
        pltpu.async_copy(feat_hbm.at[pl.ds(base_blk + g * TB, TB)], xy_v, sem)
        pltpu.async_copy(ys_hbm.at[pl.ds(base + g * T, T)], ys_v, sem)

    def wait(xy_v, ys_v, sem):
        pltpu.make_async_copy(feat_hbm.at[pl.ds(0, TB)], xy_v, sem).wait()
        pltpu.make_async_copy(ys_hbm.at[pl.ds(0, T)], ys_v, sem).wait()

    def compute(xy_v, ys_v):
        @plsc.parallel_loop(0, TB, unroll=2)
        def _blk(b):
            for grp in range(8):
                ys_f = ys_v[pl.ds(b * 128 + grp * L, L)]
                cls = ys_f.astype(jnp.int32)
                xv = xy_v[b, 0, pl.ds(grp * L, L)]
                yv = xy_v[b, 1, pl.ds(grp * L, L)]
                nv = xv * xv + yv * yv
                sidx = cls * L + lane_b[grp % NBANK]
                plsc.addupdate_scatter(acc_c, [sidx], ones)
                plsc.addupdate_scatter(acc_x, [sidx], xv)
                plsc.addupdate_scatter(acc_y, [sidx], yv)
                plsc.addupdate_scatter(acc_n, [sidx], nv)

    issue(xy0, ys0, sem0, 0)
    NPAIR = NCHUNK - (NCHUNK % 2)

    @pl.loop(0, NPAIR, step=2)
    def _chunk(g):
        @pl.when(g + 1 < NCHUNK)
        def _():
            issue(xy1, ys1, sem1, g + 1)
        wait(xy0, ys0, sem0)
        compute(xy0, ys0)

        @pl.when(g + 2 < NCHUNK)
        def _():
            issue(xy0, ys0, sem0, g + 2)
        wait(xy1, ys1, sem1)
        compute(xy1, ys1)

    if NCHUNK % 2:
        wait(xy0, ys0, sem0)
        compute(xy0, ys0)

    for k, acc in enumerate((acc_c, acc_x, acc_y, acc_n)):
        for r in range(C):
            s = pl.ds(r * L, L)
            v = acc[s]
            for q in range(1, NBANK):
                v = v + acc[pl.ds(q * C * L + r * L, L)]
            pub[s] = v
        pltpu.sync_copy(pub, out_hbm.at[wid, k])


def _run_sc(feat_blk, ys):
    mesh = plsc.VectorSubcoreMesh(core_axis_name="c", subcore_axis_name="s")
    kfn = pl.kernel(
        _sc_stats_body,
        out_type=jax.ShapeDtypeStruct((NW, STATS, C * L), jnp.float32),
        mesh=mesh,
        compiler_params=pltpu.CompilerParams(needs_layout_passes=False),
        scratch_types=[
            pltpu.VMEM((TB, 2, 128), jnp.float32),
            pltpu.VMEM((TB, 2, 128), jnp.float32),
            pltpu.VMEM((T,), jnp.float32),
            pltpu.VMEM((T,), jnp.float32),
            pltpu.VMEM((NBANK * C * L,), jnp.float32),
            pltpu.VMEM((NBANK * C * L,), jnp.float32),
            pltpu.VMEM((NBANK * C * L,), jnp.float32),
            pltpu.VMEM((NBANK * C * L,), jnp.float32),
            pltpu.VMEM((C * L,), jnp.float32),
            pltpu.SemaphoreType.DMA,
            pltpu.SemaphoreType.DMA,
        ],
    )
    return kfn(feat_blk, ys)


def _tc_stats_body(feat_ref, ys_ref, out_ref):
    i = pl.program_id(0)

    @pl.when(i == 0)
    def _():
        out_ref[...] = jnp.zeros((STATS, C * 128), jnp.float32)

    xb = feat_ref[:, 0, :]
    yb = feat_ref[:, 1, :]
    ysb = ys_ref[...]
    nb = xb * xb + yb * yb
    onesrow = jnp.ones((1, TBC), jnp.float32)
    for c in range(C):
        m = (ysb == float(c)).astype(jnp.float32)
        sl = pl.ds(c * 128, 128)
        out_ref[0:1, sl] += jnp.dot(onesrow, m,
                                    preferred_element_type=jnp.float32)
        out_ref[1:2, sl] += jnp.dot(onesrow, m * xb,
                                    preferred_element_type=jnp.float32)
        out_ref[2:3, sl] += jnp.dot(onesrow, m * yb,
                                    preferred_element_type=jnp.float32)
        out_ref[3:4, sl] += jnp.dot(onesrow, m * nb,
                                    preferred_element_type=jnp.float32)


def _run_tc(feat_blk, ys2):
    return pl.pallas_call(
        _tc_stats_body,
        grid=(NSTEP,),
        in_specs=[
            pl.BlockSpec((TBC, 2, 128), lambda i: (i, 0, 0)),
            pl.BlockSpec((TBC, 128), lambda i: (i, 0)),
        ],
        out_specs=pl.BlockSpec((STATS, C * 128), lambda i: (0, 0)),
        out_shape=jax.ShapeDtypeStruct((STATS, C * 128), jnp.float32),
    )(feat_blk, ys2)


def _fin_body(sc_ref, tc_ref, m_ref, m3_ref, ct_ref, scale_ref, out_ref):
    red = jnp.dot(jnp.sum(sc_ref[...], axis=0), m_ref[...],
                  preferred_element_type=jnp.float32)
    red = red + jnp.dot(tc_ref[...], m3_ref[...],
                        preferred_element_type=jnp.float32)
    cnt, sx, sy, sn = red[0:1], red[1:2], red[2:3], red[3:4]
    cx, cy = ct_ref[0:1], ct_ref[1:2]
    term = sn - 2.0 * (cx * sx + cy * sy) + cnt * (cx * cx + cy * cy)
    safe = jnp.where(cnt > 0, cnt, 1.0)
    per = jnp.where(cnt > 0, term / safe, 0.0)
    out_ref[...] = jnp.sum(per, axis=1, keepdims=True) * scale_ref[...]


def kernel(features, ys, center, lambdas):
    feat_blk = features.reshape(NBLK, 128, 2).transpose(0, 2, 1)
    ys2 = ys.reshape(NBLK, 128)
    sc_stats = _run_sc(feat_blk, ys)
    if B_TC:
        tc_stats = _run_tc(feat_blk, ys2)
    else:
        tc_stats = jnp.zeros((STATS, C * 128), jnp.float32)
    m = jnp.kron(jnp.eye(C, dtype=jnp.float32),
                 jnp.ones((L, 1), jnp.float32))
    m3 = jnp.kron(jnp.eye(C, dtype=jnp.float32),
                  jnp.ones((128, 1), jnp.float32))
    ct = center.T
    scale = (jnp.asarray(lambdas, jnp.float32) / 2.0 / N).reshape(1, 1)
    out = pl.pallas_call(
        _fin_body,
        out_shape=jax.ShapeDtypeStruct((1, 1), jnp.float32),
    )(sc_stats, tc_stats, m, m3, ct, scale)
    return out[0, 0]
